# baseline (speedup 1.0000x reference)
import functools

import jax
import jax.numpy as jnp
from jax import lax
from jax.experimental import pallas as pl
from jax.experimental.pallas import tpu as pltpu
from jax.experimental.pallas import tpu_sc as plsc

N = 10000
D = 128
E = 320000
NC = 2
NS = 16
NW = NC * NS
EDGES_PER_W = E // NW
CHUNK = 80
NCHUNKS = EDGES_PER_W // CHUNK
N_PAD = 10240
ROWS_PER_TILE = N_PAD // NS
ZROWS = 8


def _sc_layer(h, src3d, dst1d):
    mesh = plsc.VectorSubcoreMesh(core_axis_name="c", subcore_axis_name="s")

    @functools.partial(
        pl.kernel,
        out_type=jax.ShapeDtypeStruct((NC, N_PAD, D), jnp.float32),
        mesh=mesh,
        scratch_types=[
            pltpu.VMEM((NCHUNKS, CHUNK), jnp.int32),
            pltpu.VMEM((CHUNK,), jnp.int32),
            pltpu.VMEM((CHUNK,), jnp.int32),
            pltpu.VMEM((CHUNK,), jnp.int32),
            pltpu.VMEM((CHUNK, D), jnp.float32),
            pltpu.VMEM((CHUNK, D), jnp.float32),
            pltpu.VMEM((CHUNK, D), jnp.float32),
            pltpu.VMEM((ZROWS, D), jnp.float32),
            pltpu.VMEM_SHARED((N_PAD, D), jnp.float32),
            pltpu.SemaphoreType.DMA,
            pltpu.SemaphoreType.DMA,
            pltpu.SemaphoreType.DMA,
            pltpu.SemaphoreType.DMA,
            pltpu.SemaphoreType.DMA,
            pltpu.SemaphoreType.DMA,
            pltpu.SemaphoreType.DMA,
            pltpu.SemaphoreType.DMA,
            pltpu.SemaphoreType.DMA,
        ],
    )
    def layer_kernel(h_hbm, src_hbm, dst_hbm, out_hbm,
                     src_v, dst_a, dst_b, dst_c, rows_a, rows_b, rows_c,
                     zbuf, acc_sh,
                     sem_a, sem_b, sem_c, sem_da, sem_db, sem_dc,
                     sem_a2, sem_b2, sem_c2):
        dst_bufs = (dst_a, dst_b, dst_c)
        rows_bufs = (rows_a, rows_b, rows_c)
        gsems = (sem_a, sem_b, sem_c)
        gsems2 = (sem_a2, sem_b2, sem_c2)
        dsems = (sem_da, sem_db, sem_dc)
        c = lax.axis_index("c")
        s = lax.axis_index("s")
        wid = s * NC + c
        ebase = wid * EDGES_PER_W

        pltpu.async_copy(src_hbm.at[wid], src_v, sem_a)

        @pl.loop(0, ZROWS)
        def _(r):
            @pl.loop(0, D, step=16)
            def _(j):
                zbuf.at[r, pl.ds(j, 16)][...] = jnp.zeros((16,), jnp.float32)

        row0 = s * ROWS_PER_TILE

        @pl.loop(0, ROWS_PER_TILE // ZROWS)
        def _(k):
            pltpu.sync_copy(zbuf, acc_sh.at[pl.ds(row0 + k * ZROWS, ZROWS)])

        pltpu.make_async_copy(src_hbm.at[wid], src_v, sem_a).wait()
        plsc.subcore_barrier()

        def start_dst(j, buf, sem):
            pltpu.async_copy(dst_hbm.at[pl.ds(ebase + j * CHUNK, CHUNK)],
                             buf, sem)

        def wait_dst(j, buf, sem):
            pltpu.make_async_copy(dst_hbm.at[pl.ds(ebase + j * CHUNK, CHUNK)],
                                  buf, sem).wait()

        HC = CHUNK // 2

        def start_gather(j, buf, sem, sem2):
            pltpu.async_copy(h_hbm.at[src_v.at[j, pl.ds(0, HC)]],
                             buf.at[pl.ds(0, HC)], sem)
            pltpu.async_copy(h_hbm.at[src_v.at[j, pl.ds(HC, HC)]],
                             buf.at[pl.ds(HC, HC)], sem2)

        def wait_gather(j, buf, sem, sem2):
            pltpu.make_async_copy(h_hbm.at[src_v.at[j, pl.ds(0, HC)]],
                                  buf.at[pl.ds(0, HC)], sem).wait()
            pltpu.make_async_copy(h_hbm.at[src_v.at[j, pl.ds(HC, HC)]],
                                  buf.at[pl.ds(HC, HC)], sem2).wait()

        def scatter_add(buf, dbuf):
            pltpu.sync_copy(buf, acc_sh.at[dbuf], add=True)

        start_dst(0, dst_a, sem_da)
        start_gather(0, rows_a, sem_a, sem_a2)
        start_dst(1, dst_b, sem_db)
        start_gather(1, rows_b, sem_b, sem_b2)

        def step(m, b, do_g2, do_next):
            if do_g2:
                b2 = (b + 2) % 3
                start_dst(m + 2, dst_bufs[b2], dsems[b2])
                start_gather(m + 2, rows_bufs[b2], gsems[b2], gsems2[b2])
            wait_gather(m, rows_bufs[b], gsems[b], gsems2[b])
            wait_dst(m, dst_bufs[b], dsems[b])
            scatter_add(rows_bufs[b], dst_bufs[b])

        LOOPED = (NCHUNKS - 2) // 3 * 3

        @pl.loop(0, LOOPED, step=3)
        def _(m):
            step(m, 0, True, True)
            step(m + 1, 1, True, True)
            step(m + 2, 2, True, True)

        for m in range(LOOPED, NCHUNKS):
            step(m, m % 3, m + 2 < NCHUNKS, False)

        plsc.subcore_barrier()

        pltpu.sync_copy(acc_sh.at[pl.ds(row0, ROWS_PER_TILE)],
                        out_hbm.at[c].at[pl.ds(row0, ROWS_PER_TILE)])

    return layer_kernel(h, src3d, dst1d)


def _sum_partials(p):
    def body(p_ref, o_ref):
        o_ref[...] = p_ref[0] + p_ref[1]

    return pl.pallas_call(
        body,
        out_shape=jax.ShapeDtypeStruct((N, D), jnp.float32),
        grid=(10,),
        in_specs=[pl.BlockSpec((NC, N // 10, D), lambda i: (0, i, 0))],
        out_specs=pl.BlockSpec((N // 10, D), lambda i: (i, 0)),
    )(p)


def _head(p, W1, b1, W2, b2):
    def body(p_ref, w1_ref, b1_ref, w2_ref, b2_ref, o_ref):
        h = p_ref[0] + p_ref[1]
        h = jnp.dot(h, w1_ref[...], preferred_element_type=jnp.float32)
        h = jnp.maximum(h + b1_ref[...], 0.0)
        o = jnp.sum(h * w2_ref[...], axis=1, keepdims=True)
        o_ref[...] = o + b2_ref[0]

    return pl.pallas_call(
        body,
        out_shape=jax.ShapeDtypeStruct((N, 1), jnp.float32),
        grid=(1,),
        in_specs=[
            pl.BlockSpec((NC, N, D), lambda i: (0, 0, 0)),
            pl.BlockSpec((D, D), lambda i: (0, 0)),
            pl.BlockSpec((1, D), lambda i: (0, 0)),
            pl.BlockSpec((1, D), lambda i: (0, 0)),
            pl.BlockSpec(memory_space=pltpu.SMEM),
        ],
        out_specs=pl.BlockSpec((N, 1), lambda i: (0, 0)),
    )(p, W1, b1.reshape(1, D), W2.reshape(1, D), b2)


def kernel(x, edge_index, W1, b1, W2, b2):
    src = edge_index[0].reshape(NW, NCHUNKS, CHUNK)
    dst = edge_index[1]
    p1 = _sc_layer(x, src, dst)
    h1 = _sum_partials(p1)
    p2 = _sc_layer(h1, src, dst)
    return _head(p2, W1, b1, W2, b2)


# speedup vs baseline: 1.0068x; 1.0068x over previous
import functools

import jax
import jax.numpy as jnp
from jax import lax
from jax.experimental import pallas as pl
from jax.experimental.pallas import tpu as pltpu
from jax.experimental.pallas import tpu_sc as plsc

N = 10000
D = 128
E = 320000
NC = 2
NS = 16
NW = NC * NS
EDGES_PER_W = E // NW
CHUNK = 80
NCHUNKS = EDGES_PER_W // CHUNK
N_PAD = 10240
ROWS_PER_TILE = N_PAD // NS
ZROWS = 8


def _sc_layer(h, src3d, dst1d):
    mesh = plsc.VectorSubcoreMesh(core_axis_name="c", subcore_axis_name="s")

    @functools.partial(
        pl.kernel,
        out_type=jax.ShapeDtypeStruct((NC, N_PAD, D), jnp.float32),
        mesh=mesh,
        scratch_types=[
            pltpu.VMEM((NCHUNKS, CHUNK), jnp.int32),
            pltpu.VMEM((CHUNK,), jnp.int32),
            pltpu.VMEM((CHUNK,), jnp.int32),
            pltpu.VMEM((CHUNK,), jnp.int32),
            pltpu.VMEM((CHUNK, D), jnp.float32),
            pltpu.VMEM((CHUNK, D), jnp.float32),
            pltpu.VMEM((CHUNK, D), jnp.float32),
            pltpu.VMEM((ZROWS, D), jnp.float32),
            pltpu.VMEM_SHARED((N_PAD, D), jnp.float32),
            pltpu.SemaphoreType.DMA,
            pltpu.SemaphoreType.DMA,
            pltpu.SemaphoreType.DMA,
            pltpu.SemaphoreType.DMA,
            pltpu.SemaphoreType.DMA,
            pltpu.SemaphoreType.DMA,
        ],
    )
    def layer_kernel(h_hbm, src_hbm, dst_hbm, out_hbm,
                     src_v, dst_a, dst_b, dst_c, rows_a, rows_b, rows_c,
                     zbuf, acc_sh,
                     sem_a, sem_b, sem_c, sem_da, sem_db, sem_dc):
        dst_bufs = (dst_a, dst_b, dst_c)
        rows_bufs = (rows_a, rows_b, rows_c)
        gsems = (sem_a, sem_b, sem_c)
        dsems = (sem_da, sem_db, sem_dc)
        c = lax.axis_index("c")
        s = lax.axis_index("s")
        wid = s * NC + c
        ebase = wid * EDGES_PER_W

        pltpu.async_copy(src_hbm.at[wid], src_v, sem_a)

        @pl.loop(0, ZROWS)
        def _(r):
            @pl.loop(0, D, step=16)
            def _(j):
                zbuf.at[r, pl.ds(j, 16)][...] = jnp.zeros((16,), jnp.float32)

        row0 = s * ROWS_PER_TILE

        @pl.loop(0, ROWS_PER_TILE // ZROWS)
        def _(k):
            pltpu.sync_copy(zbuf, acc_sh.at[pl.ds(row0 + k * ZROWS, ZROWS)])

        pltpu.make_async_copy(src_hbm.at[wid], src_v, sem_a).wait()
        plsc.subcore_barrier()

        def start_dst(j, buf, sem):
            pltpu.async_copy(dst_hbm.at[pl.ds(ebase + j * CHUNK, CHUNK)],
                             buf, sem)

        def wait_dst(j, buf, sem):
            pltpu.make_async_copy(dst_hbm.at[pl.ds(ebase + j * CHUNK, CHUNK)],
                                  buf, sem).wait()

        def start_gather(j, buf, sem):
            pltpu.async_copy(h_hbm.at[src_v.at[j]], buf, sem)

        def wait_gather(j, buf, sem):
            pltpu.make_async_copy(h_hbm.at[src_v.at[j]], buf, sem).wait()

        def scatter_add(buf, dbuf):
            pltpu.sync_copy(buf, acc_sh.at[dbuf], add=True)

        start_dst(0, dst_a, sem_da)
        start_gather(0, rows_a, sem_a)
        start_dst(1, dst_b, sem_db)
        start_gather(1, rows_b, sem_b)

        def step(m, b, do_g2, do_next):
            if do_g2:
                start_dst(m + 2, dst_bufs[(b + 2) % 3], dsems[(b + 2) % 3])
                start_gather(m + 2, rows_bufs[(b + 2) % 3], gsems[(b + 2) % 3])
            wait_gather(m, rows_bufs[b], gsems[b])
            wait_dst(m, dst_bufs[b], dsems[b])
            scatter_add(rows_bufs[b], dst_bufs[b])

        LOOPED = (NCHUNKS - 2) // 3 * 3

        @pl.loop(0, LOOPED, step=3)
        def _(m):
            step(m, 0, True, True)
            step(m + 1, 1, True, True)
            step(m + 2, 2, True, True)

        for m in range(LOOPED, NCHUNKS):
            step(m, m % 3, m + 2 < NCHUNKS, False)

        plsc.subcore_barrier()

        pltpu.sync_copy(acc_sh.at[pl.ds(row0, ROWS_PER_TILE)],
                        out_hbm.at[c].at[pl.ds(row0, ROWS_PER_TILE)])

    return layer_kernel(h, src3d, dst1d)


def _sum_partials(p):
    def body(p_ref, o_ref):
        o_ref[...] = p_ref[0] + p_ref[1]

    return pl.pallas_call(
        body,
        out_shape=jax.ShapeDtypeStruct((N, D), jnp.float32),
        grid=(10,),
        in_specs=[pl.BlockSpec((NC, N // 10, D), lambda i: (0, i, 0))],
        out_specs=pl.BlockSpec((N // 10, D), lambda i: (i, 0)),
    )(p)


def _head(p, W1, b1, W2, b2):
    def body(p_ref, w1_ref, b1_ref, w2_ref, b2_ref, o_ref):
        h = p_ref[0] + p_ref[1]
        h = jnp.dot(h, w1_ref[...], preferred_element_type=jnp.float32)
        h = jnp.maximum(h + b1_ref[...], 0.0)
        o = jnp.sum(h * w2_ref[...], axis=1, keepdims=True)
        o_ref[...] = o + b2_ref[0]

    return pl.pallas_call(
        body,
        out_shape=jax.ShapeDtypeStruct((N, 1), jnp.float32),
        grid=(1,),
        in_specs=[
            pl.BlockSpec((NC, N, D), lambda i: (0, 0, 0)),
            pl.BlockSpec((D, D), lambda i: (0, 0)),
            pl.BlockSpec((1, D), lambda i: (0, 0)),
            pl.BlockSpec((1, D), lambda i: (0, 0)),
            pl.BlockSpec(memory_space=pltpu.SMEM),
        ],
        out_specs=pl.BlockSpec((N, 1), lambda i: (0, 0)),
    )(p, W1, b1.reshape(1, D), W2.reshape(1, D), b2)


def kernel(x, edge_index, W1, b1, W2, b2):
    src = edge_index[0].reshape(NW, NCHUNKS, CHUNK)
    dst = edge_index[1]
    p1 = _sc_layer(x, src, dst)
    h1 = _sum_partials(p1)
    p2 = _sc_layer(h1, src, dst)
    return _head(p2, W1, b1, W2, b2)


# speedup vs baseline: 1.0379x; 1.0309x over previous
import functools

import jax
import jax.numpy as jnp
from jax import lax
from jax.experimental import pallas as pl
from jax.experimental.pallas import tpu as pltpu
from jax.experimental.pallas import tpu_sc as plsc

N = 10000
D = 128
E = 320000
NC = 2
NS = 16
NW = NC * NS
EDGES_PER_W = E // NW
CHUNK = 80
NCHUNKS = EDGES_PER_W // CHUNK
N_PAD = 10240
ROWS_PER_TILE = N_PAD // NS
ZROWS = 8


def _sc_layer(h, src3d, dst1d):
    mesh = plsc.VectorSubcoreMesh(core_axis_name="c", subcore_axis_name="s")

    @functools.partial(
        pl.kernel,
        out_type=jax.ShapeDtypeStruct((NC, N_PAD, D), jnp.float32),
        mesh=mesh,
        scratch_types=[
            pltpu.VMEM((NCHUNKS, CHUNK), jnp.int32),
            pltpu.VMEM((CHUNK,), jnp.int32),
            pltpu.VMEM((CHUNK,), jnp.int32),
            pltpu.VMEM((CHUNK,), jnp.int32),
            pltpu.VMEM((CHUNK, D), jnp.float32),
            pltpu.VMEM((CHUNK, D), jnp.float32),
            pltpu.VMEM((CHUNK, D), jnp.float32),
            pltpu.VMEM_SHARED((N_PAD, D), jnp.float32),
            pltpu.SemaphoreType.DMA,
            pltpu.SemaphoreType.DMA,
            pltpu.SemaphoreType.DMA,
            pltpu.SemaphoreType.DMA,
            pltpu.SemaphoreType.DMA,
            pltpu.SemaphoreType.DMA,
        ],
    )
    def layer_kernel(h_hbm, src_hbm, dst_hbm, out_hbm,
                     src_v, dst_a, dst_b, dst_c, rows_a, rows_b, rows_c,
                     acc_sh,
                     sem_a, sem_b, sem_c, sem_da, sem_db, sem_dc):
        dst_bufs = (dst_a, dst_b, dst_c)
        rows_bufs = (rows_a, rows_b, rows_c)
        gsems = (sem_a, sem_b, sem_c)
        dsems = (sem_da, sem_db, sem_dc)
        c = lax.axis_index("c")
        s = lax.axis_index("s")
        wid = s * NC + c
        ebase = wid * EDGES_PER_W

        pltpu.async_copy(src_hbm.at[wid], src_v, sem_a)

        @pl.loop(0, CHUNK)
        def _(r):
            @pl.loop(0, D, step=16)
            def _(j):
                rows_a.at[r, pl.ds(j, 16)][...] = jnp.zeros((16,), jnp.float32)

        row0 = s * ROWS_PER_TILE

        @pl.loop(0, ROWS_PER_TILE // CHUNK)
        def _(k):
            pltpu.sync_copy(rows_a, acc_sh.at[pl.ds(row0 + k * CHUNK, CHUNK)])

        pltpu.make_async_copy(src_hbm.at[wid], src_v, sem_a).wait()
        plsc.subcore_barrier()

        def start_dst(j, buf, sem):
            pltpu.async_copy(dst_hbm.at[pl.ds(ebase + j * CHUNK, CHUNK)],
                             buf, sem)

        def wait_dst(j, buf, sem):
            pltpu.make_async_copy(dst_hbm.at[pl.ds(ebase + j * CHUNK, CHUNK)],
                                  buf, sem).wait()

        def start_gather(j, buf, sem):
            pltpu.async_copy(h_hbm.at[src_v.at[j]], buf, sem)

        def wait_gather(j, buf, sem):
            pltpu.make_async_copy(h_hbm.at[src_v.at[j]], buf, sem).wait()

        def scatter_add(buf, dbuf):
            pltpu.sync_copy(buf, acc_sh.at[dbuf], add=True)

        start_dst(0, dst_a, sem_da)
        start_gather(0, rows_a, sem_a)
        start_dst(1, dst_b, sem_db)
        start_gather(1, rows_b, sem_b)

        def step(m, b, do_g2, do_next):
            if do_g2:
                start_dst(m + 2, dst_bufs[(b + 2) % 3], dsems[(b + 2) % 3])
                start_gather(m + 2, rows_bufs[(b + 2) % 3], gsems[(b + 2) % 3])
            wait_gather(m, rows_bufs[b], gsems[b])
            wait_dst(m, dst_bufs[b], dsems[b])
            scatter_add(rows_bufs[b], dst_bufs[b])

        LOOPED = (NCHUNKS - 2) // 3 * 3

        @pl.loop(0, LOOPED, step=3)
        def _(m):
            step(m, 0, True, True)
            step(m + 1, 1, True, True)
            step(m + 2, 2, True, True)

        for m in range(LOOPED, NCHUNKS):
            step(m, m % 3, m + 2 < NCHUNKS, False)

        plsc.subcore_barrier()

        pltpu.sync_copy(acc_sh.at[pl.ds(row0, ROWS_PER_TILE)],
                        out_hbm.at[c].at[pl.ds(row0, ROWS_PER_TILE)])

    return layer_kernel(h, src3d, dst1d)


def _sum_partials(p):
    def body(p_ref, o_ref):
        o_ref[...] = p_ref[0] + p_ref[1]

    return pl.pallas_call(
        body,
        out_shape=jax.ShapeDtypeStruct((N, D), jnp.float32),
        grid=(10,),
        in_specs=[pl.BlockSpec((NC, N // 10, D), lambda i: (0, i, 0))],
        out_specs=pl.BlockSpec((N // 10, D), lambda i: (i, 0)),
    )(p)


def _head(p, W1, b1, W2, b2):
    def body(p_ref, w1_ref, b1_ref, w2_ref, b2_ref, o_ref):
        h = p_ref[0] + p_ref[1]
        h = jnp.dot(h, w1_ref[...], preferred_element_type=jnp.float32)
        h = jnp.maximum(h + b1_ref[...], 0.0)
        o = jnp.sum(h * w2_ref[...], axis=1, keepdims=True)
        o_ref[...] = o + b2_ref[0]

    return pl.pallas_call(
        body,
        out_shape=jax.ShapeDtypeStruct((N, 1), jnp.float32),
        grid=(1,),
        in_specs=[
            pl.BlockSpec((NC, N, D), lambda i: (0, 0, 0)),
            pl.BlockSpec((D, D), lambda i: (0, 0)),
            pl.BlockSpec((1, D), lambda i: (0, 0)),
            pl.BlockSpec((1, D), lambda i: (0, 0)),
            pl.BlockSpec(memory_space=pltpu.SMEM),
        ],
        out_specs=pl.BlockSpec((N, 1), lambda i: (0, 0)),
    )(p, W1, b1.reshape(1, D), W2.reshape(1, D), b2)


def kernel(x, edge_index, W1, b1, W2, b2):
    src = edge_index[0].reshape(NW, NCHUNKS, CHUNK)
    dst = edge_index[1]
    p1 = _sc_layer(x, src, dst)
    h1 = _sum_partials(p1)
    p2 = _sc_layer(h1, src, dst)
    return _head(p2, W1, b1, W2, b2)
